# baseline (speedup 1.0000x reference)
import functools

import jax
import jax.numpy as jnp
from jax import lax
from jax.experimental import pallas as pl
from jax.experimental.pallas import tpu as pltpu
from jax.experimental.pallas import tpu_sc as plsc

_N_NBR = 160000
_N_NODES = 10000
_F = 128
_H = 256
_E = 256

_NC = 2
_NS = 16
_NW = _NC * _NS

_SPLIT = 0
_SC_ROWS = _N_NBR - _SPLIT
_WROWS = _SC_ROWS // _NW
_CH = 40
_NCHUNK = _WROWS // _CH
_UNROLL = 5

_RBLK = 2000
_DBLK = 2000
_ND = _N_NODES // _DBLK

assert _SC_ROWS % _NW == 0 and _WROWS % _CH == 0 and _CH % 8 == 0
assert (_WROWS % 8 == 0) and (_CH % _UNROLL == 0) and _NCHUNK >= 3


def _sc_reduce_body(l1_hbm, l2_hbm, o1_hbm, o2_hbm,
                    b1a, b1b, b2a, b2b, a1v, a2v,
                    s1a, s1b, s2a, s2b):
    wid = lax.axis_index("s") * _NC + lax.axis_index("c")
    base = _SPLIT + wid * _WROWS

    bufs1 = (b1a, b1b)
    bufs2 = (b2a, b2b)
    sems1 = (s1a, s1b)
    sems2 = (s2a, s2b)

    def cp1(k, slot):
        return pltpu.make_async_copy(
            l1_hbm.at[pl.ds(base + k * _CH, _CH)], bufs1[slot], sems1[slot])

    def cp2(k, slot):
        return pltpu.make_async_copy(
            l2_hbm.at[pl.ds(base + k * _CH, _CH)], bufs2[slot], sems2[slot])

    z = jnp.zeros((16,), jnp.float32)

    def accum(buf, ncc):
        def rb(rr, a):
            for u in range(_UNROLL):
                r = rr * _UNROLL + u
                a = tuple(a[j] + buf[r, pl.ds(16 * j, 16)] for j in range(ncc))
            return a
        return lax.fori_loop(0, _CH // _UNROLL, rb, tuple(z for _ in range(ncc)))

    cp1(0, 0).start()
    cp2(0, 0).start()
    cp1(1, 1).start()
    cp2(1, 1).start()

    for j in range(_F // 16):
        a1v[pl.ds(16 * j, 16)] = z
    for j in range(_H // 16):
        a2v[pl.ds(16 * j, 16)] = z

    def chunk_step(k, slot, with_start):
        cp1(k, slot).wait()
        cp2(k, slot).wait()
        if with_start:
            cp1(k + 2, slot).start()
            cp2(k + 2, slot).start()
        p1 = accum(bufs1[slot], _F // 16)
        p2 = accum(bufs2[slot], _H // 16)
        for j in range(_F // 16):
            a1v[pl.ds(16 * j, 16)] += p1[j]
        for j in range(_H // 16):
            a2v[pl.ds(16 * j, 16)] += p2[j]

    def body(k, carry):
        @pl.when(k % 2 == 0)
        def _():
            chunk_step(k, 0, True)

        @pl.when(k % 2 == 1)
        def _():
            chunk_step(k, 1, True)

        return carry

    lax.fori_loop(0, _NCHUNK - 2, body, 0)
    chunk_step(_NCHUNK - 2, (_NCHUNK - 2) % 2, False)
    chunk_step(_NCHUNK - 1, (_NCHUNK - 1) % 2, False)

    pltpu.sync_copy(a1v, o1_hbm.at[wid])
    pltpu.sync_copy(a2v, o2_hbm.at[wid])


def _sc_reduce(l1, l2):
    f32 = jnp.float32
    mesh = plsc.VectorSubcoreMesh(core_axis_name="c", subcore_axis_name="s")
    fn = functools.partial(
        pl.kernel,
        mesh=mesh,
        out_type=[
            jax.ShapeDtypeStruct((_NW, _F), f32),
            jax.ShapeDtypeStruct((_NW, _H), f32),
        ],
        scratch_types=[
            pltpu.VMEM((_CH, _F), f32),
            pltpu.VMEM((_CH, _F), f32),
            pltpu.VMEM((_CH, _H), f32),
            pltpu.VMEM((_CH, _H), f32),
            pltpu.VMEM((_F,), f32),
            pltpu.VMEM((_H,), f32),
            pltpu.SemaphoreType.DMA,
            pltpu.SemaphoreType.DMA,
            pltpu.SemaphoreType.DMA,
            pltpu.SemaphoreType.DMA,
        ],
    )(_sc_reduce_body)
    return fn(l1, l2)


def _ln_relu(x, g, b, eps=1e-5):
    mu = jnp.mean(x, axis=-1, keepdims=True)
    var = jnp.mean((x - mu) ** 2, axis=-1, keepdims=True)
    y = (x - mu) / jnp.sqrt(var + eps) * g + b
    return jnp.maximum(y, 0.0)


def _dense_body(nf_ref, o1_ref, o2_ref, ws1_ref, wn1_ref, c1b_ref, g1_ref,
                be1_ref, ws2_ref, wn2_ref, c2b_ref, g2_ref, be2_ref,
                out_ref, ssq, h2s):
    i = pl.program_id(0)

    @pl.when(i == 0)
    def _():
        ssq[...] = jnp.zeros_like(ssq)

    @pl.when(i < _ND)
    def _():
        j = i
        inv = 1.0 / _N_NBR
        agg1 = jnp.sum(o1_ref[...], axis=0, keepdims=True) * inv
        agg2 = jnp.sum(o2_ref[...], axis=0, keepdims=True) * inv
        c1 = jnp.dot(agg1, wn1_ref[...], preferred_element_type=jnp.float32) + c1b_ref[...]
        out1 = jnp.dot(nf_ref[...], ws1_ref[...], preferred_element_type=jnp.float32) + c1
        h1 = _ln_relu(out1, g1_ref[...], be1_ref[...])
        c2 = jnp.dot(agg2, wn2_ref[...], preferred_element_type=jnp.float32) + c2b_ref[...]
        out2 = jnp.dot(h1, ws2_ref[...], preferred_element_type=jnp.float32) + c2
        h2 = _ln_relu(out2, g2_ref[...], be2_ref[...])
        h2s[pl.ds(j * _DBLK, _DBLK), :] = h2
        ssq[...] += jnp.sum(h2 * h2, axis=0, keepdims=True)

    @pl.when(i >= _ND)
    def _():
        j = i - _ND
        norm = jnp.sqrt(ssq[...])
        out_ref[...] = h2s[pl.ds(j * _DBLK, _DBLK), :] / jnp.maximum(norm, 1e-12)


def _tc_head_reduce_body(l1_ref, l2_ref, s1_ref, s2_ref):
    i = pl.program_id(0)

    @pl.when(i == 0)
    def _():
        s1_ref[...] = jnp.zeros_like(s1_ref)
        s2_ref[...] = jnp.zeros_like(s2_ref)

    s1_ref[...] += jnp.sum(l1_ref[...], axis=0, keepdims=True)
    s2_ref[...] += jnp.sum(l2_ref[...], axis=0, keepdims=True)


@jax.jit
def kernel(node_feat, neighbor_feats_l1, neighbor_feats_l2, W_self1, b_self1,
           W_nbr1, b_nbr1, g1, be1, W_self2, b_self2, W_nbr2, b_nbr2, g2, be2):
    f32 = jnp.float32
    c1b = (b_self1 + b_nbr1).reshape(1, _H)
    c2b = (b_self2 + b_nbr2).reshape(1, _E)

    o1, o2 = _sc_reduce(neighbor_feats_l1, neighbor_feats_l2)

    if _SPLIT > 0:
        s1h, s2h = pl.pallas_call(
            _tc_head_reduce_body,
            grid=(_SPLIT // _RBLK,),
            in_specs=[
                pl.BlockSpec((_RBLK, _F), lambda i: (i, 0)),
                pl.BlockSpec((_RBLK, _H), lambda i: (i, 0)),
            ],
            out_specs=[
                pl.BlockSpec((1, _F), lambda i: (0, 0)),
                pl.BlockSpec((1, _H), lambda i: (0, 0)),
            ],
            out_shape=[
                jax.ShapeDtypeStruct((1, _F), f32),
                jax.ShapeDtypeStruct((1, _H), f32),
            ],
        )(neighbor_feats_l1, neighbor_feats_l2)
        o1_all = jnp.concatenate([o1, s1h], axis=0)
        o2_all = jnp.concatenate([o2, s2h], axis=0)
    else:
        o1_all, o2_all = o1, o2

    npart = o1_all.shape[0]

    def _clamp(lo, x, hi):
        return jnp.minimum(jnp.maximum(x, lo), hi)

    out = pl.pallas_call(
        _dense_body,
        grid=(2 * _ND,),
        in_specs=[
            pl.BlockSpec((_DBLK, _F), lambda i: (_clamp(0, i, _ND - 1), 0)),
            pl.BlockSpec((npart, _F), lambda i: (0, 0)),
            pl.BlockSpec((npart, _H), lambda i: (0, 0)),
            pl.BlockSpec((_F, _H), lambda i: (0, 0)),
            pl.BlockSpec((_F, _H), lambda i: (0, 0)),
            pl.BlockSpec((1, _H), lambda i: (0, 0)),
            pl.BlockSpec((1, _H), lambda i: (0, 0)),
            pl.BlockSpec((1, _H), lambda i: (0, 0)),
            pl.BlockSpec((_H, _E), lambda i: (0, 0)),
            pl.BlockSpec((_H, _E), lambda i: (0, 0)),
            pl.BlockSpec((1, _E), lambda i: (0, 0)),
            pl.BlockSpec((1, _E), lambda i: (0, 0)),
            pl.BlockSpec((1, _E), lambda i: (0, 0)),
        ],
        out_specs=pl.BlockSpec((_DBLK, _E), lambda i: (_clamp(0, i - _ND, _ND - 1), 0)),
        out_shape=jax.ShapeDtypeStruct((_N_NODES, _E), f32),
        scratch_shapes=[
            pltpu.VMEM((1, _E), f32),
            pltpu.VMEM((_N_NODES, _E), f32),
        ],
    )(node_feat, o1_all, o2_all, W_self1, W_nbr1, c1b,
      g1.reshape(1, _H), be1.reshape(1, _H), W_self2, W_nbr2, c2b,
      g2.reshape(1, _E), be2.reshape(1, _E))

    return out


# speedup vs baseline: 1.1062x; 1.1062x over previous
import functools

import jax
import jax.numpy as jnp
from jax import lax
from jax.experimental import pallas as pl
from jax.experimental.pallas import tpu as pltpu
from jax.experimental.pallas import tpu_sc as plsc

_N_NBR = 160000
_N_NODES = 10000
_F = 128
_H = 256
_E = 256

_NC = 2
_NS = 16
_NW = _NC * _NS

_SPLIT = 64000
_SC_ROWS = _N_NBR - _SPLIT
_WROWS = _SC_ROWS // _NW
_CH = 40
_NCHUNK = _WROWS // _CH
_UNROLL = 5

_RBLK = 2000
_DBLK = 2000
_ND = _N_NODES // _DBLK

assert _SC_ROWS % _NW == 0 and _WROWS % _CH == 0 and _CH % 8 == 0
assert (_WROWS % 8 == 0) and (_CH % _UNROLL == 0) and _NCHUNK >= 3


def _sc_reduce_body(l1_hbm, l2_hbm, o1_hbm, o2_hbm,
                    b1a, b1b, b2a, b2b, a1v, a2v,
                    s1a, s1b, s2a, s2b):
    wid = lax.axis_index("s") * _NC + lax.axis_index("c")
    base = _SPLIT + wid * _WROWS

    bufs1 = (b1a, b1b)
    bufs2 = (b2a, b2b)
    sems1 = (s1a, s1b)
    sems2 = (s2a, s2b)

    def cp1(k, slot):
        return pltpu.make_async_copy(
            l1_hbm.at[pl.ds(base + k * _CH, _CH)], bufs1[slot], sems1[slot])

    def cp2(k, slot):
        return pltpu.make_async_copy(
            l2_hbm.at[pl.ds(base + k * _CH, _CH)], bufs2[slot], sems2[slot])

    z = jnp.zeros((16,), jnp.float32)

    def accum(buf, ncc):
        def rb(rr, a):
            for u in range(_UNROLL):
                r = rr * _UNROLL + u
                a = tuple(a[j] + buf[r, pl.ds(16 * j, 16)] for j in range(ncc))
            return a
        return lax.fori_loop(0, _CH // _UNROLL, rb, tuple(z for _ in range(ncc)))

    cp1(0, 0).start()
    cp2(0, 0).start()
    cp1(1, 1).start()
    cp2(1, 1).start()

    for j in range(_F // 16):
        a1v[pl.ds(16 * j, 16)] = z
    for j in range(_H // 16):
        a2v[pl.ds(16 * j, 16)] = z

    def chunk_step(k, slot, with_start):
        cp1(k, slot).wait()
        cp2(k, slot).wait()
        if with_start:
            cp1(k + 2, slot).start()
            cp2(k + 2, slot).start()
        p1 = accum(bufs1[slot], _F // 16)
        p2 = accum(bufs2[slot], _H // 16)
        for j in range(_F // 16):
            a1v[pl.ds(16 * j, 16)] += p1[j]
        for j in range(_H // 16):
            a2v[pl.ds(16 * j, 16)] += p2[j]

    def body(k, carry):
        @pl.when(k % 2 == 0)
        def _():
            chunk_step(k, 0, True)

        @pl.when(k % 2 == 1)
        def _():
            chunk_step(k, 1, True)

        return carry

    lax.fori_loop(0, _NCHUNK - 2, body, 0)
    chunk_step(_NCHUNK - 2, (_NCHUNK - 2) % 2, False)
    chunk_step(_NCHUNK - 1, (_NCHUNK - 1) % 2, False)

    pltpu.sync_copy(a1v, o1_hbm.at[wid])
    pltpu.sync_copy(a2v, o2_hbm.at[wid])


def _sc_reduce(l1, l2):
    f32 = jnp.float32
    mesh = plsc.VectorSubcoreMesh(core_axis_name="c", subcore_axis_name="s")
    fn = functools.partial(
        pl.kernel,
        mesh=mesh,
        out_type=[
            jax.ShapeDtypeStruct((_NW, _F), f32),
            jax.ShapeDtypeStruct((_NW, _H), f32),
        ],
        scratch_types=[
            pltpu.VMEM((_CH, _F), f32),
            pltpu.VMEM((_CH, _F), f32),
            pltpu.VMEM((_CH, _H), f32),
            pltpu.VMEM((_CH, _H), f32),
            pltpu.VMEM((_F,), f32),
            pltpu.VMEM((_H,), f32),
            pltpu.SemaphoreType.DMA,
            pltpu.SemaphoreType.DMA,
            pltpu.SemaphoreType.DMA,
            pltpu.SemaphoreType.DMA,
        ],
    )(_sc_reduce_body)
    return fn(l1, l2)


def _ln_relu(x, g, b, eps=1e-5):
    mu = jnp.mean(x, axis=-1, keepdims=True)
    var = jnp.mean((x - mu) ** 2, axis=-1, keepdims=True)
    y = (x - mu) / jnp.sqrt(var + eps) * g + b
    return jnp.maximum(y, 0.0)


def _dense_body(nf_ref, o1_ref, o2_ref, s1h_ref, s2h_ref, ws1_ref, wn1_ref,
                c1b_ref, g1_ref, be1_ref, ws2_ref, wn2_ref, c2b_ref, g2_ref,
                be2_ref, out_ref, ssq, h2s):
    i = pl.program_id(0)

    @pl.when(i == 0)
    def _():
        ssq[...] = jnp.zeros_like(ssq)

    @pl.when(i < _ND)
    def _():
        j = i
        inv = 1.0 / _N_NBR
        agg1 = (jnp.sum(o1_ref[...], axis=0, keepdims=True) + s1h_ref[...]) * inv
        agg2 = (jnp.sum(o2_ref[...], axis=0, keepdims=True) + s2h_ref[...]) * inv
        c1 = jnp.dot(agg1, wn1_ref[...], preferred_element_type=jnp.float32) + c1b_ref[...]
        out1 = jnp.dot(nf_ref[...], ws1_ref[...], preferred_element_type=jnp.float32) + c1
        h1 = _ln_relu(out1, g1_ref[...], be1_ref[...])
        c2 = jnp.dot(agg2, wn2_ref[...], preferred_element_type=jnp.float32) + c2b_ref[...]
        out2 = jnp.dot(h1, ws2_ref[...], preferred_element_type=jnp.float32) + c2
        h2 = _ln_relu(out2, g2_ref[...], be2_ref[...])
        h2s[pl.ds(j * _DBLK, _DBLK), :] = h2
        ssq[...] += jnp.sum(h2 * h2, axis=0, keepdims=True)

    @pl.when(i >= _ND)
    def _():
        j = i - _ND
        norm = jnp.sqrt(ssq[...])
        out_ref[...] = h2s[pl.ds(j * _DBLK, _DBLK), :] / jnp.maximum(norm, 1e-12)


def _tc_head_reduce_body(l1_ref, l2_ref, s1_ref, s2_ref):
    i = pl.program_id(0)

    @pl.when(i == 0)
    def _():
        s1_ref[...] = jnp.zeros_like(s1_ref)
        s2_ref[...] = jnp.zeros_like(s2_ref)

    s1_ref[...] += jnp.sum(l1_ref[...], axis=0, keepdims=True)
    s2_ref[...] += jnp.sum(l2_ref[...], axis=0, keepdims=True)


@jax.jit
def kernel(node_feat, neighbor_feats_l1, neighbor_feats_l2, W_self1, b_self1,
           W_nbr1, b_nbr1, g1, be1, W_self2, b_self2, W_nbr2, b_nbr2, g2, be2):
    f32 = jnp.float32
    c1b = (b_self1 + b_nbr1).reshape(1, _H)
    c2b = (b_self2 + b_nbr2).reshape(1, _E)

    o1, o2 = _sc_reduce(neighbor_feats_l1, neighbor_feats_l2)

    if _SPLIT > 0:
        s1h, s2h = pl.pallas_call(
            _tc_head_reduce_body,
            grid=(_SPLIT // _RBLK,),
            in_specs=[
                pl.BlockSpec((_RBLK, _F), lambda i: (i, 0)),
                pl.BlockSpec((_RBLK, _H), lambda i: (i, 0)),
            ],
            out_specs=[
                pl.BlockSpec((1, _F), lambda i: (0, 0)),
                pl.BlockSpec((1, _H), lambda i: (0, 0)),
            ],
            out_shape=[
                jax.ShapeDtypeStruct((1, _F), f32),
                jax.ShapeDtypeStruct((1, _H), f32),
            ],
        )(neighbor_feats_l1, neighbor_feats_l2)
    else:
        s1h = jnp.zeros((1, _F), f32)
        s2h = jnp.zeros((1, _H), f32)

    npart = _NW

    def _clamp(lo, x, hi):
        return jnp.minimum(jnp.maximum(x, lo), hi)

    out = pl.pallas_call(
        _dense_body,
        grid=(2 * _ND,),
        in_specs=[
            pl.BlockSpec((_DBLK, _F), lambda i: (_clamp(0, i, _ND - 1), 0)),
            pl.BlockSpec((npart, _F), lambda i: (0, 0)),
            pl.BlockSpec((npart, _H), lambda i: (0, 0)),
            pl.BlockSpec((1, _F), lambda i: (0, 0)),
            pl.BlockSpec((1, _H), lambda i: (0, 0)),
            pl.BlockSpec((_F, _H), lambda i: (0, 0)),
            pl.BlockSpec((_F, _H), lambda i: (0, 0)),
            pl.BlockSpec((1, _H), lambda i: (0, 0)),
            pl.BlockSpec((1, _H), lambda i: (0, 0)),
            pl.BlockSpec((1, _H), lambda i: (0, 0)),
            pl.BlockSpec((_H, _E), lambda i: (0, 0)),
            pl.BlockSpec((_H, _E), lambda i: (0, 0)),
            pl.BlockSpec((1, _E), lambda i: (0, 0)),
            pl.BlockSpec((1, _E), lambda i: (0, 0)),
            pl.BlockSpec((1, _E), lambda i: (0, 0)),
        ],
        out_specs=pl.BlockSpec((_DBLK, _E), lambda i: (_clamp(0, i - _ND, _ND - 1), 0)),
        out_shape=jax.ShapeDtypeStruct((_N_NODES, _E), f32),
        scratch_shapes=[
            pltpu.VMEM((1, _E), f32),
            pltpu.VMEM((_N_NODES, _E), f32),
        ],
    )(node_feat, o1, o2, s1h, s2h, W_self1, W_nbr1, c1b,
      g1.reshape(1, _H), be1.reshape(1, _H), W_self2, W_nbr2, c2b,
      g2.reshape(1, _E), be2.reshape(1, _E))

    return out
